# baseline (speedup 1.0000x reference)
import functools

import jax
import jax.numpy as jnp
from jax import lax
from jax.experimental import pallas as pl
from jax.experimental.pallas import tpu as pltpu
from jax.experimental.pallas import tpu_sc as plsc

_NUM_EMBEDDINGS = 1024
_EMBED_DIM = 768
_NUM_HEADS = 4
_DH = _EMBED_DIM // _NUM_HEADS
_COMMITMENT_COST = 0.25

_BLOCK = 512
_N = 9216

_NW = 32
_WPH = _NW // _NUM_HEADS
_ROWS_PER_W = _N // _WPH
_CHUNK = 128
_NCHUNK = _ROWS_PER_W // _CHUNK
_NBUF = 3


def _dist_kernel(x_ref, w_ref, codes_ref, loss_ref, b_scr):
    @pl.when(pl.program_id(0) == 0)
    def _():
        for h in range(_NUM_HEADS):
            wh = w_ref[h]
            b_scr[h] = jnp.sum(wh * wh, axis=1)[None, :]

    x = x_ref[...]
    acc = jnp.zeros((), dtype=jnp.float32)
    code_iota = jax.lax.broadcasted_iota(jnp.int32, (1, _NUM_EMBEDDINGS), 1)
    for h in range(_NUM_HEADS):
        xh = x[:, h * _DH:(h + 1) * _DH]
        wh = w_ref[h]
        m = jax.lax.dot_general(
            xh, wh, (((1,), (1,)), ((), ())),
            preferred_element_type=jnp.float32)
        a = jnp.sum(xh * xh, axis=1, keepdims=True)
        d = (a + b_scr[h]) - 2.0 * m
        dmin = jnp.min(d, axis=1, keepdims=True)
        idx = jnp.min(
            jnp.where(d == dmin, code_iota, _NUM_EMBEDDINGS),
            axis=1).astype(jnp.int32)
        codes_ref[h, :] = idx
        acc = acc + jnp.sum(dmin)
    loss_ref[...] = acc.reshape(1, 1, 1)


def _gather_kernel(w_hbm, codes_hbm, out_hbm, idx_v, rows_v,
                   g0, g1, g2, s0, s1, s2, isem):
    gsems = [g0, g1, g2]
    ssems = [s0, s1, s2]
    wid = lax.axis_index("s") * 2 + lax.axis_index("c")
    h = wid // _WPH
    i0 = (wid % _WPH) * _ROWS_PER_W
    icopies = [
        pltpu.async_copy(
            codes_hbm.at[h, pl.ds(i0 + j * _CHUNK, _CHUNK)],
            idx_v.at[j], isem)
        for j in range(_NCHUNK)
    ]
    for c in icopies:
        c.wait()
    gcopies = [None] * _NCHUNK
    for j in range(_NBUF):
        gcopies[j] = pltpu.async_copy(
            w_hbm.at[h].at[idx_v.at[j]], rows_v.at[j], gsems[j])
    for j in range(_NCHUNK):
        b = j % _NBUF
        gcopies[j].wait()
        out_cp = pltpu.async_copy(
            rows_v.at[b],
            out_hbm.at[pl.ds(i0 + j * _CHUNK, _CHUNK), h], ssems[b])
        if j + _NBUF < _NCHUNK:
            out_cp.wait()
            gcopies[j + _NBUF] = pltpu.async_copy(
                w_hbm.at[h].at[idx_v.at[j + _NBUF]], rows_v.at[b],
                gsems[b])
        else:
            out_cp.wait()


@jax.jit
def kernel(inputs, emb_weights):
    input_shape = inputs.shape
    x = inputs.reshape(_N, _EMBED_DIM)
    nblocks = _N // _BLOCK

    codes, loss_parts = pl.pallas_call(
        _dist_kernel,
        grid=(nblocks,),
        in_specs=[
            pl.BlockSpec((_BLOCK, _EMBED_DIM), lambda i: (i, 0)),
            pl.BlockSpec((_NUM_HEADS, _NUM_EMBEDDINGS, _DH),
                         lambda i: (0, 0, 0)),
        ],
        out_specs=[
            pl.BlockSpec((_NUM_HEADS, _BLOCK), lambda i: (0, i)),
            pl.BlockSpec((1, 1, 1), lambda i: (i, 0, 0)),
        ],
        out_shape=[
            jax.ShapeDtypeStruct((_NUM_HEADS, _N), jnp.int32),
            jax.ShapeDtypeStruct((nblocks, 1, 1), jnp.float32),
        ],
        scratch_shapes=[pltpu.VMEM((_NUM_HEADS, 1, _NUM_EMBEDDINGS),
                                   jnp.float32)],
        compiler_params=pltpu.CompilerParams(
            dimension_semantics=("arbitrary",)),
    )(x, emb_weights)

    mesh = plsc.VectorSubcoreMesh(core_axis_name="c", subcore_axis_name="s")
    gather = functools.partial(
        pl.kernel,
        mesh=mesh,
        out_type=jax.ShapeDtypeStruct((_N, _NUM_HEADS, _DH), jnp.float32),
        scratch_types=[
            pltpu.VMEM((_NCHUNK, _CHUNK), jnp.int32),
            pltpu.VMEM((_NBUF, _CHUNK, _DH), jnp.float32),
        ] + [pltpu.SemaphoreType.DMA] * 7,
        compiler_params=pltpu.CompilerParams(use_tc_tiling_on_sc=False),
    )(_gather_kernel)
    q = gather(emb_weights, codes)

    numel = _N * _EMBED_DIM
    loss = jnp.sum(loss_parts) * (_COMMITMENT_COST / numel)
    quantized = q.reshape(input_shape)
    vq_codes = codes.reshape(_NUM_HEADS, _N, 1)
    return loss, quantized, vq_codes


# speedup vs baseline: 1.7782x; 1.7782x over previous
import functools

import jax
import jax.numpy as jnp
from jax.experimental import pallas as pl
from jax.experimental.pallas import tpu as pltpu

_NUM_EMBEDDINGS = 1024
_EMBED_DIM = 768
_NUM_HEADS = 4
_DH = _EMBED_DIM // _NUM_HEADS
_COMMITMENT_COST = 0.25

_BLOCK = 256


def _vq_kernel(x_ref, w_ref, q_ref, codes_ref, loss_ref, b_scr):
    @pl.when(pl.program_id(0) == 0)
    def _():
        for h in range(_NUM_HEADS):
            wh = w_ref[h]
            b_scr[h] = jnp.sum(wh * wh, axis=1)[None, :]

    x = x_ref[...]
    acc = jnp.zeros((), dtype=jnp.float32)
    idx_cols = []
    iota_f = jax.lax.broadcasted_iota(
        jnp.int32, (1, _NUM_EMBEDDINGS), 1).astype(jnp.float32)
    for h in range(_NUM_HEADS):
        xh = x[:, h * _DH:(h + 1) * _DH]
        wh = w_ref[h]
        m = jax.lax.dot_general(
            xh, wh, (((1,), (1,)), ((), ())),
            preferred_element_type=jnp.float32)
        a = jnp.sum(xh * xh, axis=1, keepdims=True)
        d = (a + b_scr[h]) - 2.0 * m
        dmin = jnp.min(d, axis=1, keepdims=True)
        idxf = jnp.min(
            jnp.where(d == dmin, iota_f, jnp.float32(_NUM_EMBEDDINGS)),
            axis=1, keepdims=True)
        idx_cols.append(idxf)
        onehot = (iota_f == idxf).astype(jnp.float32)
        qh = jax.lax.dot_general(
            onehot, wh, (((1,), (0,)), ((), ())),
            preferred_element_type=jnp.float32)
        q_ref[:, h * _DH:(h + 1) * _DH] = qh
        acc = acc + jnp.sum(dmin)
    codes_ref[...] = jnp.concatenate(idx_cols, axis=1).astype(jnp.int32)
    loss_ref[...] = acc.reshape(1, 1, 1)


@jax.jit
def kernel(inputs, emb_weights):
    input_shape = inputs.shape
    n = input_shape[0] * input_shape[1]
    x = inputs.reshape(n, _EMBED_DIM)
    nblocks = n // _BLOCK

    q, codes, loss_parts = pl.pallas_call(
        _vq_kernel,
        grid=(nblocks,),
        in_specs=[
            pl.BlockSpec((_BLOCK, _EMBED_DIM), lambda i: (i, 0)),
            pl.BlockSpec((_NUM_HEADS, _NUM_EMBEDDINGS, _DH),
                         lambda i: (0, 0, 0)),
        ],
        out_specs=[
            pl.BlockSpec((_BLOCK, _EMBED_DIM), lambda i: (i, 0)),
            pl.BlockSpec((_BLOCK, _NUM_HEADS), lambda i: (i, 0)),
            pl.BlockSpec((1, 1, 1), lambda i: (i, 0, 0)),
        ],
        out_shape=[
            jax.ShapeDtypeStruct((n, _EMBED_DIM), jnp.float32),
            jax.ShapeDtypeStruct((n, _NUM_HEADS), jnp.int32),
            jax.ShapeDtypeStruct((nblocks, 1, 1), jnp.float32),
        ],
        scratch_shapes=[pltpu.VMEM((_NUM_HEADS, 1, _NUM_EMBEDDINGS),
                                   jnp.float32)],
        compiler_params=pltpu.CompilerParams(
            dimension_semantics=("arbitrary",)),
    )(x, emb_weights)

    numel = n * _EMBED_DIM
    loss = jnp.sum(loss_parts) * (_COMMITMENT_COST / numel)
    quantized = q.reshape(input_shape)
    vq_codes = codes.T[:, :, None]
    return loss, quantized, vq_codes


# speedup vs baseline: 1.7868x; 1.0048x over previous
import functools

import jax
import jax.numpy as jnp
from jax.experimental import pallas as pl
from jax.experimental.pallas import tpu as pltpu

_NUM_EMBEDDINGS = 1024
_EMBED_DIM = 768
_NUM_HEADS = 4
_DH = _EMBED_DIM // _NUM_HEADS
_COMMITMENT_COST = 0.25

_BLOCK = 256


def _vq_kernel(x_ref, w_ref, q_ref, codes_ref, loss_ref, b_scr, w2_scr):
    @pl.when(pl.program_id(0) == 0)
    def _():
        for h in range(_NUM_HEADS):
            wh = w_ref[h]
            b_scr[h] = jnp.sum(wh * wh, axis=1)[None, :]
            w2_scr[h] = wh + wh

    x = x_ref[...]
    acc = jnp.zeros((), dtype=jnp.float32)
    idx_cols = []
    iota_f = jax.lax.broadcasted_iota(
        jnp.int32, (1, _NUM_EMBEDDINGS), 1).astype(jnp.float32)
    for h in range(_NUM_HEADS):
        xh = x[:, h * _DH:(h + 1) * _DH]
        wh = w_ref[h]
        m2 = jax.lax.dot_general(
            xh, w2_scr[h], (((1,), (1,)), ((), ())),
            preferred_element_type=jnp.float32)
        a = jnp.sum(xh * xh, axis=1, keepdims=True)
        d = (a + b_scr[h]) - m2
        dmin = jnp.min(d, axis=1, keepdims=True)
        idxf = jnp.min(
            jnp.where(d == dmin, iota_f, jnp.float32(_NUM_EMBEDDINGS)),
            axis=1, keepdims=True)
        idx_cols.append(idxf)
        onehot = (iota_f == idxf).astype(jnp.float32)
        qh = jax.lax.dot_general(
            onehot, wh, (((1,), (0,)), ((), ())),
            preferred_element_type=jnp.float32)
        q_ref[:, h * _DH:(h + 1) * _DH] = qh
        acc = acc + jnp.sum(dmin)
    codes_ref[...] = jnp.concatenate(idx_cols, axis=1).astype(jnp.int32)
    loss_ref[...] = acc.reshape(1, 1, 1)


@jax.jit
def kernel(inputs, emb_weights):
    input_shape = inputs.shape
    n = input_shape[0] * input_shape[1]
    x = inputs.reshape(n, _EMBED_DIM)
    nblocks = n // _BLOCK

    q, codes, loss_parts = pl.pallas_call(
        _vq_kernel,
        grid=(nblocks,),
        in_specs=[
            pl.BlockSpec((_BLOCK, _EMBED_DIM), lambda i: (i, 0)),
            pl.BlockSpec((_NUM_HEADS, _NUM_EMBEDDINGS, _DH),
                         lambda i: (0, 0, 0)),
        ],
        out_specs=[
            pl.BlockSpec((_BLOCK, _EMBED_DIM), lambda i: (i, 0)),
            pl.BlockSpec((_BLOCK, _NUM_HEADS), lambda i: (i, 0)),
            pl.BlockSpec((1, 1, 1), lambda i: (i, 0, 0)),
        ],
        out_shape=[
            jax.ShapeDtypeStruct((n, _EMBED_DIM), jnp.float32),
            jax.ShapeDtypeStruct((n, _NUM_HEADS), jnp.int32),
            jax.ShapeDtypeStruct((nblocks, 1, 1), jnp.float32),
        ],
        scratch_shapes=[
            pltpu.VMEM((_NUM_HEADS, 1, _NUM_EMBEDDINGS), jnp.float32),
            pltpu.VMEM((_NUM_HEADS, _NUM_EMBEDDINGS, _DH), jnp.float32),
        ],
        compiler_params=pltpu.CompilerParams(
            dimension_semantics=("arbitrary",)),
    )(x, emb_weights)

    numel = n * _EMBED_DIM
    loss = jnp.sum(loss_parts) * (_COMMITMENT_COST / numel)
    quantized = q.reshape(input_shape)
    vq_codes = codes.T[:, :, None]
    return loss, quantized, vq_codes
